# baseline (speedup 1.0000x reference)
import functools

import jax
import jax.numpy as jnp
from jax import lax
from jax.experimental import pallas as pl
from jax.experimental.pallas import tpu as pltpu
from jax.experimental.pallas import tpu_sc as plsc

UST_DIM = 8
_NC, _NS, _LANES = 2, 16, 16
_NW = _NC * _NS


def _nn_scale_kernel(scale_ref):
    F = scale_ref.shape[-1]
    qi = jax.lax.broadcasted_iota(jnp.int32, (F, F), 0)
    pj = jax.lax.broadcasted_iota(jnp.int32, (F, F), 1)
    diff = (qi - pj).astype(jnp.float32)
    dists = jnp.float32(UST_DIM) * (diff * diff)
    idx = jnp.argmin(dists, axis=1)
    scale_ref[...] = ((idx.astype(jnp.float32) + 1.0) / jnp.float32(F))[None, :]


def _fused_kernel(x_ref, o_ref, scale_ref):
    F = x_ref.shape[1]

    @pl.when(pl.program_id(0) == 0)
    def _compute_scale():
        _nn_scale_kernel(scale_ref)

    o_ref[...] = x_ref[...] * scale_ref[...]


def _make_sc_mul(B, F, sc_rows, row_base):
    rows_per_w = sc_rows // _NW
    chunk = 64
    n_chunks = rows_per_w // chunk
    mesh = plsc.VectorSubcoreMesh(core_axis_name="c", subcore_axis_name="s")

    @functools.partial(
        pl.kernel,
        mesh=mesh,
        out_type=jax.ShapeDtypeStruct((sc_rows, F), jnp.float32),
        scratch_types=[
            pltpu.VMEM((chunk, F), jnp.float32),
            pltpu.VMEM((F,), jnp.float32),
        ],
    )
    def sc_mul(x_hbm, scale_hbm, out_hbm, buf_v, scale_v):
        wid = lax.axis_index("s") * _NC + lax.axis_index("c")
        base = row_base + wid * rows_per_w
        obase = wid * rows_per_w
        pltpu.sync_copy(scale_hbm, scale_v)
        for c in range(n_chunks):
            pltpu.sync_copy(x_hbm.at[pl.ds(base + c * chunk, chunk)], buf_v)

            def rbody(r, _, buf_v=buf_v, scale_v=scale_v):
                for f in range(F // _LANES):
                    sl = pl.ds(f * _LANES, _LANES)
                    buf_v[r, sl] = buf_v[r, sl] * scale_v[sl]
                return 0

            lax.fori_loop(0, chunk, rbody, 0)
            pltpu.sync_copy(buf_v, out_hbm.at[pl.ds(obase + c * chunk, chunk)])

    return sc_mul


def kernel(inputs):
    B, F = inputs.shape
    SC_ROWS = 4096
    TC_ROWS = B - SC_ROWS

    scale = pl.pallas_call(
        _nn_scale_kernel,
        out_shape=jax.ShapeDtypeStruct((1, F), jnp.float32),
    )()

    BLK = 2048
    out_tc = pl.pallas_call(
        _fused_kernel,
        grid=(TC_ROWS // BLK,),
        in_specs=[pl.BlockSpec((BLK, F), lambda i: (i, 0))],
        out_specs=pl.BlockSpec((BLK, F), lambda i: (i, 0)),
        out_shape=jax.ShapeDtypeStruct((TC_ROWS, F), inputs.dtype),
        scratch_shapes=[pltpu.VMEM((1, F), jnp.float32)],
        compiler_params=pltpu.CompilerParams(
            dimension_semantics=("arbitrary",),
        ),
    )(inputs)

    out_sc = _make_sc_mul(B, F, SC_ROWS, TC_ROWS)(inputs, scale.reshape(F))
    return jnp.concatenate([out_tc, out_sc], axis=0)


# speedup vs baseline: 1.1752x; 1.1752x over previous
import functools

import jax
import jax.numpy as jnp
from jax import lax
from jax.experimental import pallas as pl
from jax.experimental.pallas import tpu as pltpu
from jax.experimental.pallas import tpu_sc as plsc

UST_DIM = 8
_NC, _NS, _LANES = 2, 16, 16
_NW = _NC * _NS


def _nn_scale_kernel(scale_ref):
    F = scale_ref.shape[-1]
    qi = jax.lax.broadcasted_iota(jnp.int32, (F, F), 0)
    pj = jax.lax.broadcasted_iota(jnp.int32, (F, F), 1)
    diff = (qi - pj).astype(jnp.float32)
    dists = jnp.float32(UST_DIM) * (diff * diff)
    idx = jnp.argmin(dists, axis=1)
    scale_ref[...] = ((idx.astype(jnp.float32) + 1.0) / jnp.float32(F))[None, :]


def _fused_kernel(x_ref, o_ref, scale_ref):
    F = x_ref.shape[1]

    @pl.when(pl.program_id(0) == 0)
    def _compute_scale():
        _nn_scale_kernel(scale_ref)

    o_ref[...] = x_ref[...] * scale_ref[...]


def _make_sc_mul(B, F, sc_rows, row_base):
    rows_per_w = sc_rows // _NW
    chunk = 16
    n_chunks = rows_per_w // chunk
    mesh = plsc.VectorSubcoreMesh(
        core_axis_name="c", subcore_axis_name="s", num_cores=_NC
    )

    @functools.partial(
        pl.kernel,
        mesh=mesh,
        out_type=jax.ShapeDtypeStruct((sc_rows, F), jnp.float32),
        scratch_types=[
            pltpu.VMEM((2, chunk, F), jnp.float32),
            pltpu.VMEM((2, chunk, F), jnp.float32),
            pltpu.VMEM((F,), jnp.float32),
        ]
        + [pltpu.SemaphoreType.DMA] * 4,
    )
    def sc_mul(x_hbm, scale_hbm, out_hbm, ibuf_v, obuf_v, scale_v, *sems):
        in_sems = sems[:2]
        out_sems = sems[2:]
        wid = lax.axis_index("s") * _NC + lax.axis_index("c")
        base = row_base + wid * rows_per_w
        obase = wid * rows_per_w
        pltpu.sync_copy(scale_hbm, scale_v)

        def start_in(c):
            return pltpu.async_copy(
                x_hbm.at[pl.ds(base + c * chunk, chunk)],
                ibuf_v.at[c % 2],
                in_sems[c % 2],
            )

        def compute(c):
            bi = c % 2

            def fbody(f, _):
                sl = pl.ds(f * _LANES, _LANES)
                sv = scale_v[sl]

                def rbody(r, _):
                    for u in range(4):
                        row = r * 4 + u
                        obuf_v[bi, row, sl] = ibuf_v[bi, row, sl] * sv
                    return 0

                lax.fori_loop(0, chunk // 4, rbody, 0)
                return 0

            lax.fori_loop(0, F // _LANES, fbody, 0)

        def start_out(c):
            return pltpu.async_copy(
                obuf_v.at[c % 2],
                out_hbm.at[pl.ds(obase + c * chunk, chunk)],
                out_sems[c % 2],
            )

        in_copies = [None] * n_chunks
        out_copies = [None] * n_chunks
        in_copies[0] = start_in(0)
        for c in range(n_chunks):
            in_copies[c].wait()
            if c + 1 < n_chunks:
                in_copies[c + 1] = start_in(c + 1)
            if c >= 2:
                out_copies[c - 2].wait()
            compute(c)
            out_copies[c] = start_out(c)
        for c in range(max(0, n_chunks - 2), n_chunks):
            out_copies[c].wait()

    return sc_mul


def kernel(inputs):
    B, F = inputs.shape
    SC_ROWS = 4096
    TC_ROWS = B - SC_ROWS

    scale = pl.pallas_call(
        _nn_scale_kernel,
        out_shape=jax.ShapeDtypeStruct((1, F), jnp.float32),
    )()

    BLK = 2048
    out_tc = pl.pallas_call(
        _fused_kernel,
        grid=(TC_ROWS // BLK,),
        in_specs=[pl.BlockSpec((BLK, F), lambda i: (i, 0))],
        out_specs=pl.BlockSpec((BLK, F), lambda i: (i, 0)),
        out_shape=jax.ShapeDtypeStruct((TC_ROWS, F), inputs.dtype),
        scratch_shapes=[pltpu.VMEM((1, F), jnp.float32)],
        compiler_params=pltpu.CompilerParams(
            dimension_semantics=("arbitrary",),
        ),
    )(inputs)

    out_sc = _make_sc_mul(B, F, SC_ROWS, TC_ROWS)(inputs, scale.reshape(F))
    return jnp.concatenate([out_tc, out_sc], axis=0)


# speedup vs baseline: 2.0762x; 1.7667x over previous
import functools

import jax
import jax.numpy as jnp
from jax import lax
from jax.experimental import pallas as pl
from jax.experimental.pallas import tpu as pltpu
from jax.experimental.pallas import tpu_sc as plsc

UST_DIM = 8
_NC, _NS, _LANES = 2, 16, 16
_NW = _NC * _NS


def _make_sc_scale(F):
    q_per_w = F // _NW
    n_qv = q_per_w // _LANES
    mesh = plsc.VectorSubcoreMesh(
        core_axis_name="c", subcore_axis_name="s", num_cores=_NC
    )

    @functools.partial(
        pl.kernel,
        mesh=mesh,
        out_type=jax.ShapeDtypeStruct((F,), jnp.float32),
        scratch_types=[pltpu.VMEM((q_per_w,), jnp.float32)],
    )
    def scale_sc(out_hbm, buf_v):
        wid = lax.axis_index("s") * _NC + lax.axis_index("c")
        base = wid * q_per_w
        lane = lax.iota(jnp.int32, _LANES)
        for qv in range(n_qv):
            qf = (base + qv * _LANES + lane).astype(jnp.float32)

            def nbody(step, carry, qf=qf):
                mind, mini = carry
                for u in range(8):
                    n = step * 8 + u
                    diff = qf - n.astype(jnp.float32)
                    dist = jnp.float32(UST_DIM) * (diff * diff)
                    better = dist < mind
                    mind = jnp.where(better, dist, mind)
                    mini = jnp.where(better, n, mini)
                return mind, mini

            mind0 = jnp.full((_LANES,), jnp.float32(3.4e38))
            mini0 = jnp.zeros((_LANES,), jnp.int32)
            _, mini = lax.fori_loop(0, F // 8, nbody, (mind0, mini0))
            buf_v[pl.ds(qv * _LANES, _LANES)] = (
                mini.astype(jnp.float32) + 1.0
            ) / jnp.float32(F)
        pltpu.sync_copy(buf_v, out_hbm.at[pl.ds(base, q_per_w)])

    return scale_sc


def _fused_kernel(x_ref, o_ref, scale_ref):
    F = x_ref.shape[1]

    @pl.when(pl.program_id(0) == 0)
    def _compute_scale():
        qi = jax.lax.broadcasted_iota(jnp.int32, (F, F), 0)
        pj = jax.lax.broadcasted_iota(jnp.int32, (F, F), 1)
        diff = (qi - pj).astype(jnp.float32)
        dists = jnp.float32(UST_DIM) * (diff * diff)
        idx = jnp.argmin(dists, axis=1)
        scale_ref[...] = ((idx.astype(jnp.float32) + 1.0) / jnp.float32(F))[None, :]

    o_ref[...] = x_ref[...] * scale_ref[...]


def _mul2_kernel(dst_ref, x_ref, scale_ref, o_ref):
    del dst_ref
    o_ref[...] = x_ref[...] * scale_ref[...]


def kernel(inputs):
    B, F = inputs.shape
    BLK = 2048
    B1 = 8192

    scale_sc = _make_sc_scale(F)()

    out_a = pl.pallas_call(
        _fused_kernel,
        grid=(B1 // BLK,),
        in_specs=[pl.BlockSpec((BLK, F), lambda i: (i, 0))],
        out_specs=pl.BlockSpec((BLK, F), lambda i: (i, 0)),
        out_shape=jax.ShapeDtypeStruct((B, F), inputs.dtype),
        scratch_shapes=[pltpu.VMEM((1, F), jnp.float32)],
        compiler_params=pltpu.CompilerParams(
            dimension_semantics=("arbitrary",),
        ),
    )(inputs)

    n2 = (B - B1) // BLK
    off = B1 // BLK
    out = pl.pallas_call(
        _mul2_kernel,
        grid=(n2,),
        in_specs=[
            pl.BlockSpec(memory_space=pl.ANY),
            pl.BlockSpec((BLK, F), lambda i, off=off: (off + i, 0)),
            pl.BlockSpec((1, F), lambda i: (0, 0)),
        ],
        out_specs=pl.BlockSpec((BLK, F), lambda i, off=off: (off + i, 0)),
        out_shape=jax.ShapeDtypeStruct((B, F), inputs.dtype),
        input_output_aliases={0: 0},
        compiler_params=pltpu.CompilerParams(
            dimension_semantics=("arbitrary",),
        ),
    )(out_a, inputs, scale_sc.reshape(1, F))
    return out


# speedup vs baseline: 2.0812x; 1.0024x over previous
import functools

import jax
import jax.numpy as jnp
from jax import lax
from jax.experimental import pallas as pl
from jax.experimental.pallas import tpu as pltpu
from jax.experimental.pallas import tpu_sc as plsc

UST_DIM = 8
_NC, _NS, _LANES = 2, 16, 16
_NW = _NC * _NS


def _make_sc_scale(F):
    q_per_w = F // _NW
    n_qv = q_per_w // _LANES
    mesh = plsc.VectorSubcoreMesh(
        core_axis_name="c", subcore_axis_name="s", num_cores=_NC
    )

    @functools.partial(
        pl.kernel,
        mesh=mesh,
        out_type=jax.ShapeDtypeStruct((F,), jnp.float32),
        scratch_types=[pltpu.VMEM((q_per_w,), jnp.float32)],
    )
    def scale_sc(out_hbm, buf_v):
        wid = lax.axis_index("s") * _NC + lax.axis_index("c")
        base = wid * q_per_w
        lane = lax.iota(jnp.int32, _LANES)
        for qv in range(n_qv):
            qf = (base + qv * _LANES + lane).astype(jnp.float32)

            def nbody(step, carry, qf=qf):
                mind, mini = carry
                for u in range(2):
                    n = step * 2 + u
                    diff = qf - n.astype(jnp.float32)
                    dist = jnp.float32(UST_DIM) * (diff * diff)
                    better = dist < mind
                    mind = jnp.where(better, dist, mind)
                    mini = jnp.where(better, n, mini)
                return mind, mini

            mind0 = jnp.full((_LANES,), jnp.float32(3.4e38))
            mini0 = jnp.zeros((_LANES,), jnp.int32)
            _, mini = lax.fori_loop(0, F // 2, nbody, (mind0, mini0))
            buf_v[pl.ds(qv * _LANES, _LANES)] = (
                mini.astype(jnp.float32) + 1.0
            ) / jnp.float32(F)
        pltpu.sync_copy(buf_v, out_hbm.at[pl.ds(base, q_per_w)])

    return scale_sc


def _fused_kernel(x_ref, o_ref, scale_ref):
    F = x_ref.shape[1]

    @pl.when(pl.program_id(0) == 0)
    def _compute_scale():
        qi = jax.lax.broadcasted_iota(jnp.int32, (F, F), 0)
        pj = jax.lax.broadcasted_iota(jnp.int32, (F, F), 1)
        diff = (qi - pj).astype(jnp.float32)
        dists = jnp.float32(UST_DIM) * (diff * diff)
        idx = jnp.argmin(dists, axis=1)
        scale_ref[...] = ((idx.astype(jnp.float32) + 1.0) / jnp.float32(F))[None, :]

    o_ref[...] = x_ref[...] * scale_ref[...]


def _mul2_kernel(dst_ref, x_ref, scale_ref, o_ref):
    del dst_ref
    o_ref[...] = x_ref[...] * scale_ref[...]


def kernel(inputs):
    B, F = inputs.shape
    BLK = 2048
    B1 = 8192

    scale_sc = _make_sc_scale(F)()

    out_a = pl.pallas_call(
        _fused_kernel,
        grid=(B1 // BLK,),
        in_specs=[pl.BlockSpec((BLK, F), lambda i: (i, 0))],
        out_specs=pl.BlockSpec((BLK, F), lambda i: (i, 0)),
        out_shape=jax.ShapeDtypeStruct((B, F), inputs.dtype),
        scratch_shapes=[pltpu.VMEM((1, F), jnp.float32)],
        compiler_params=pltpu.CompilerParams(
            dimension_semantics=("arbitrary",),
        ),
    )(inputs)

    n2 = (B - B1) // BLK
    off = B1 // BLK
    out = pl.pallas_call(
        _mul2_kernel,
        grid=(n2,),
        in_specs=[
            pl.BlockSpec(memory_space=pl.ANY),
            pl.BlockSpec((BLK, F), lambda i, off=off: (off + i, 0)),
            pl.BlockSpec((1, F), lambda i: (0, 0)),
        ],
        out_specs=pl.BlockSpec((BLK, F), lambda i, off=off: (off + i, 0)),
        out_shape=jax.ShapeDtypeStruct((B, F), inputs.dtype),
        input_output_aliases={0: 0},
        compiler_params=pltpu.CompilerParams(
            dimension_semantics=("arbitrary",),
        ),
    )(out_a, inputs, scale_sc.reshape(1, F))
    return out


# speedup vs baseline: 2.1329x; 1.0248x over previous
import functools

import jax
import jax.numpy as jnp
from jax import lax
from jax.experimental import pallas as pl
from jax.experimental.pallas import tpu as pltpu
from jax.experimental.pallas import tpu_sc as plsc

UST_DIM = 8
_NC, _NS, _LANES = 2, 16, 16
_NW = _NC * _NS


def _make_sc_scale(F):
    num_cores = 1
    q_per_w = F // (num_cores * _NS)
    n_qv = q_per_w // _LANES
    mesh = plsc.VectorSubcoreMesh(
        core_axis_name="c", subcore_axis_name="s", num_cores=num_cores
    )

    @functools.partial(
        pl.kernel,
        mesh=mesh,
        out_type=jax.ShapeDtypeStruct((F,), jnp.float32),
        scratch_types=[pltpu.VMEM((q_per_w,), jnp.float32)],
    )
    def scale_sc(out_hbm, buf_v):
        wid = lax.axis_index("s") * num_cores + lax.axis_index("c")
        base = wid * q_per_w
        lane = lax.iota(jnp.int32, _LANES)
        for qv in range(n_qv):
            qf = (base + qv * _LANES + lane).astype(jnp.float32)

            def nbody(step, carry, qf=qf):
                mind, mini = carry
                for u in range(2):
                    n = step * 2 + u
                    diff = qf - n.astype(jnp.float32)
                    dist = jnp.float32(UST_DIM) * (diff * diff)
                    better = dist < mind
                    mind = jnp.where(better, dist, mind)
                    mini = jnp.where(better, n, mini)
                return mind, mini

            mind0 = jnp.full((_LANES,), jnp.float32(3.4e38))
            mini0 = jnp.zeros((_LANES,), jnp.int32)
            _, mini = lax.fori_loop(0, F // 2, nbody, (mind0, mini0))
            buf_v[pl.ds(qv * _LANES, _LANES)] = (
                mini.astype(jnp.float32) + 1.0
            ) / jnp.float32(F)
        pltpu.sync_copy(buf_v, out_hbm.at[pl.ds(base, q_per_w)])

    return scale_sc


def _fused_kernel(x_ref, o_ref, scale_ref):
    F = x_ref.shape[1]

    @pl.when(pl.program_id(0) == 0)
    def _compute_scale():
        qi = jax.lax.broadcasted_iota(jnp.int32, (F, F), 0)
        pj = jax.lax.broadcasted_iota(jnp.int32, (F, F), 1)
        diff = (qi - pj).astype(jnp.float32)
        dists = jnp.float32(UST_DIM) * (diff * diff)
        idx = jnp.argmin(dists, axis=1)
        scale_ref[...] = ((idx.astype(jnp.float32) + 1.0) / jnp.float32(F))[None, :]

    o_ref[...] = x_ref[...] * scale_ref[...]


def _mul2_kernel(dst_ref, x_ref, scale_ref, o_ref):
    del dst_ref
    o_ref[...] = x_ref[...] * scale_ref[...]


def kernel(inputs):
    B, F = inputs.shape
    BLK = 2048
    B1 = 8192

    scale_sc = _make_sc_scale(F)()

    out_a = pl.pallas_call(
        _fused_kernel,
        grid=(B1 // BLK,),
        in_specs=[pl.BlockSpec((BLK, F), lambda i: (i, 0))],
        out_specs=pl.BlockSpec((BLK, F), lambda i: (i, 0)),
        out_shape=jax.ShapeDtypeStruct((B, F), inputs.dtype),
        scratch_shapes=[pltpu.VMEM((1, F), jnp.float32)],
        compiler_params=pltpu.CompilerParams(
            dimension_semantics=("arbitrary",),
        ),
    )(inputs)

    n2 = (B - B1) // BLK
    off = B1 // BLK
    out = pl.pallas_call(
        _mul2_kernel,
        grid=(n2,),
        in_specs=[
            pl.BlockSpec(memory_space=pl.ANY),
            pl.BlockSpec((BLK, F), lambda i, off=off: (off + i, 0)),
            pl.BlockSpec((1, F), lambda i: (0, 0)),
        ],
        out_specs=pl.BlockSpec((BLK, F), lambda i, off=off: (off + i, 0)),
        out_shape=jax.ShapeDtypeStruct((B, F), inputs.dtype),
        input_output_aliases={0: 0},
        compiler_params=pltpu.CompilerParams(
            dimension_semantics=("arbitrary",),
        ),
    )(out_a, inputs, scale_sc.reshape(1, F))
    return out


# speedup vs baseline: 2.1354x; 1.0012x over previous
import functools

import jax
import jax.numpy as jnp
from jax import lax
from jax.experimental import pallas as pl
from jax.experimental.pallas import tpu as pltpu
from jax.experimental.pallas import tpu_sc as plsc

UST_DIM = 8
_NS, _LANES = 16, 16


def _make_sc_scale(F):
    num_cores = 1
    q_per_w = F // (num_cores * _NS)
    n_qv = q_per_w // _LANES
    mesh = plsc.VectorSubcoreMesh(
        core_axis_name="c", subcore_axis_name="s", num_cores=num_cores
    )

    @functools.partial(
        pl.kernel,
        mesh=mesh,
        out_type=jax.ShapeDtypeStruct((F,), jnp.float32),
        scratch_types=[pltpu.VMEM((q_per_w,), jnp.float32)],
    )
    def scale_sc(out_hbm, buf_v):
        wid = lax.axis_index("s") * num_cores + lax.axis_index("c")
        base = wid * q_per_w
        lane = lax.iota(jnp.int32, _LANES)
        for qv in range(n_qv):
            qf = (base + qv * _LANES + lane).astype(jnp.float32)

            def nbody(step, carry, qf=qf):
                mind, mini = carry
                for u in range(2):
                    n = step * 2 + u
                    diff = qf - n.astype(jnp.float32)
                    dist = jnp.float32(UST_DIM) * (diff * diff)
                    better = dist < mind
                    mind = jnp.where(better, dist, mind)
                    mini = jnp.where(better, n, mini)
                return mind, mini

            mind0 = jnp.full((_LANES,), jnp.float32(3.4e38))
            mini0 = jnp.zeros((_LANES,), jnp.int32)
            _, mini = lax.fori_loop(0, F // 2, nbody, (mind0, mini0))
            buf_v[pl.ds(qv * _LANES, _LANES)] = (
                mini.astype(jnp.float32) + 1.0
            ) / jnp.float32(F)
        pltpu.sync_copy(buf_v, out_hbm.at[pl.ds(base, q_per_w)])

    return scale_sc


def _fused_kernel(x_ref, o_ref, scale_ref):
    F = x_ref.shape[1]

    @pl.when(pl.program_id(0) == 0)
    def _compute_scale():
        qi = jax.lax.broadcasted_iota(jnp.int32, (F, F), 0)
        pj = jax.lax.broadcasted_iota(jnp.int32, (F, F), 1)
        diff = (qi - pj).astype(jnp.float32)
        dists = jnp.float32(UST_DIM) * (diff * diff)
        idx = jnp.argmin(dists, axis=1)
        scale_ref[...] = ((idx.astype(jnp.float32) + 1.0) / jnp.float32(F))[None, :]

    o_ref[...] = x_ref[...] * scale_ref[...]


def _mul2_kernel(dst_ref, x_ref, scale_ref, o_ref):
    del dst_ref
    o_ref[...] = x_ref[...] * scale_ref[...]


def kernel(inputs):
    B, F = inputs.shape
    BLK = 2048
    B1 = 8192

    scale_sc = _make_sc_scale(F)()

    out_a = pl.pallas_call(
        _fused_kernel,
        grid=(B1 // BLK,),
        in_specs=[pl.BlockSpec((BLK, F), lambda i: (i, 0))],
        out_specs=pl.BlockSpec((BLK, F), lambda i: (i, 0)),
        out_shape=jax.ShapeDtypeStruct((B, F), inputs.dtype),
        scratch_shapes=[pltpu.VMEM((1, F), jnp.float32)],
        compiler_params=pltpu.CompilerParams(
            dimension_semantics=("arbitrary",),
        ),
    )(inputs)

    n2 = (B - B1) // BLK
    off = B1 // BLK
    out = pl.pallas_call(
        _mul2_kernel,
        grid=(n2,),
        in_specs=[
            pl.BlockSpec(memory_space=pl.ANY),
            pl.BlockSpec((BLK, F), lambda i, off=off: (off + i, 0)),
            pl.BlockSpec((1, F), lambda i: (0, 0)),
        ],
        out_specs=pl.BlockSpec((BLK, F), lambda i, off=off: (off + i, 0)),
        out_shape=jax.ShapeDtypeStruct((B, F), inputs.dtype),
        input_output_aliases={0: 0},
        compiler_params=pltpu.CompilerParams(
            dimension_semantics=("arbitrary",),
        ),
    )(out_a, inputs, scale_sc.reshape(1, F))
    return out
